# baseline (speedup 1.0000x reference)
import jax
import jax.numpy as jnp
from jax.experimental import pallas as pl
from jax.experimental.pallas import tpu as pltpu

Q = 1024
K = 100000
D = 64
BK = 1000
NB = K // BK
CH = 8
NCH = BK // CH


def _nn_kernel(x_ref, idx_ref, xsq_ref, dist_out, idx_out,
               m_ref, minv_ref, mini_ref):
    j = pl.program_id(0)

    @pl.when(j == 0)
    def _init():
        minv_ref[...] = jnp.full((CH, Q), jnp.inf, jnp.float32)
        mini_ref[...] = jnp.zeros((CH, Q), jnp.int32)

    m_ref[...] = jax.lax.dot_general(
        idx_ref[...], x_ref[...], (((1,), (1,)), ((), ())),
        preferred_element_type=jnp.float32)

    xsq = xsq_ref[...]
    iota = jax.lax.broadcasted_iota(jnp.int32, (CH, Q), 0)
    base = j * BK

    def body(c, _):
        r0 = c * CH
        mslab = m_ref[pl.ds(r0, CH), :]
        bslab = idx_ref[pl.ds(r0, CH), :]
        isq = jnp.sum(bslab * bslab, axis=1, keepdims=True)
        t = (xsq + isq) - 2.0 * mslab
        rid = (base + r0) + iota
        better = t < minv_ref[...]
        minv_ref[...] = jnp.where(better, t, minv_ref[...])
        mini_ref[...] = jnp.where(better, rid, mini_ref[...])
        return _

    jax.lax.fori_loop(0, NCH, body, None)

    @pl.when(j == NB - 1)
    def _emit():
        minv = minv_ref[...]
        mini = mini_ref[...]
        gmin = jnp.min(minv, axis=0, keepdims=True)
        cand = jnp.where(minv == gmin, mini, K)
        gidx = jnp.min(cand, axis=0, keepdims=True)
        dist_out[...] = jnp.maximum(gmin, 0.0)
        idx_out[...] = gidx


@jax.jit
def kernel(x, index):
    x_sq = jnp.sum(x * x, axis=1)[None, :]
    dist, idx = pl.pallas_call(
        _nn_kernel,
        grid=(NB,),
        in_specs=[
            pl.BlockSpec((Q, D), lambda j: (0, 0)),
            pl.BlockSpec((BK, D), lambda j: (j, 0)),
            pl.BlockSpec((1, Q), lambda j: (0, 0)),
        ],
        out_specs=[
            pl.BlockSpec((1, Q), lambda j: (0, 0)),
            pl.BlockSpec((1, Q), lambda j: (0, 0)),
        ],
        out_shape=[
            jax.ShapeDtypeStruct((1, Q), jnp.float32),
            jax.ShapeDtypeStruct((1, Q), jnp.int32),
        ],
        scratch_shapes=[
            pltpu.VMEM((BK, Q), jnp.float32),
            pltpu.VMEM((CH, Q), jnp.float32),
            pltpu.VMEM((CH, Q), jnp.int32),
        ],
        compiler_params=pltpu.CompilerParams(
            dimension_semantics=("arbitrary",),
        ),
    )(x, index, x_sq)
    return dist.reshape(Q, 1), idx.reshape(Q, 1)


# speedup vs baseline: 2.9951x; 2.9951x over previous
import jax
import jax.numpy as jnp
from jax.experimental import pallas as pl
from jax.experimental.pallas import tpu as pltpu

Q = 1024
K = 100000
D = 64
BK = 200
NB = K // BK


def _nn_kernel(x_ref, idx_ref, xsq_ref, dist_out, idx_out,
               minv_ref, mblk_ref):
    j = pl.program_id(0)

    blk = idx_ref[...]
    m = jax.lax.dot_general(
        blk, x_ref[...], (((1,), (1,)), ((), ())),
        preferred_element_type=jnp.float32)

    isq = jnp.sum(blk * blk, axis=1, keepdims=True)
    t = (xsq_ref[...] + isq) - 2.0 * m

    @pl.when(j == 0)
    def _init():
        minv_ref[...] = t
        mblk_ref[...] = jnp.zeros((BK, Q), jnp.int32)

    @pl.when(j > 0)
    def _update():
        better = t < minv_ref[...]
        minv_ref[...] = jnp.where(better, t, minv_ref[...])
        mblk_ref[...] = jnp.where(better, j, mblk_ref[...])

    @pl.when(j == NB - 1)
    def _emit():
        minv = minv_ref[...]
        rows = jax.lax.broadcasted_iota(jnp.int32, (BK, Q), 0)
        rid = mblk_ref[...] * BK + rows
        gmin = jnp.min(minv, axis=0, keepdims=True)
        cand = jnp.where(minv == gmin, rid, K)
        gidx = jnp.min(cand, axis=0, keepdims=True)
        dist_out[...] = jnp.maximum(gmin, 0.0)
        idx_out[...] = gidx


@jax.jit
def kernel(x, index):
    x_sq = jnp.sum(x * x, axis=1)[None, :]
    dist, idx = pl.pallas_call(
        _nn_kernel,
        grid=(NB,),
        in_specs=[
            pl.BlockSpec((Q, D), lambda j: (0, 0)),
            pl.BlockSpec((BK, D), lambda j: (j, 0)),
            pl.BlockSpec((1, Q), lambda j: (0, 0)),
        ],
        out_specs=[
            pl.BlockSpec((1, Q), lambda j: (0, 0)),
            pl.BlockSpec((1, Q), lambda j: (0, 0)),
        ],
        out_shape=[
            jax.ShapeDtypeStruct((1, Q), jnp.float32),
            jax.ShapeDtypeStruct((1, Q), jnp.int32),
        ],
        scratch_shapes=[
            pltpu.VMEM((BK, Q), jnp.float32),
            pltpu.VMEM((BK, Q), jnp.int32),
        ],
        compiler_params=pltpu.CompilerParams(
            dimension_semantics=("arbitrary",),
        ),
    )(x, index, x_sq)
    return dist.reshape(Q, 1), idx.reshape(Q, 1)


# speedup vs baseline: 8.0342x; 2.6825x over previous
import jax
import jax.numpy as jnp
from jax.experimental import pallas as pl
from jax.experimental.pallas import tpu as pltpu

Q = 1024
K = 100000
D = 64
BK = 1000
NB = K // BK
NG = BK // 8


def _nn_kernel(x_ref, idx_ref, xsq_ref, dist_out, idx_out,
               minv_ref, grp_ref):
    j = pl.program_id(0)

    @pl.when(j == 0)
    def _init():
        minv_ref[...] = jnp.full((8, Q), jnp.inf, jnp.float32)
        grp_ref[...] = jnp.zeros((8, Q), jnp.int32)

    blk = idx_ref[...]
    m = jax.lax.dot_general(
        blk, x_ref[...], (((1,), (1,)), ((), ())),
        preferred_element_type=jnp.float32)

    isq = jnp.sum(blk * blk, axis=1, keepdims=True)
    t = (xsq_ref[...] + isq) - 2.0 * m

    minv8 = t[0:8, :]
    gbest = jnp.zeros((8, Q), jnp.int32)
    for g in range(1, NG):
        tg = t[g * 8:(g + 1) * 8, :]
        better = tg < minv8
        minv8 = jnp.where(better, tg, minv8)
        gbest = jnp.where(better, g, gbest)

    better = minv8 < minv_ref[...]
    minv_ref[...] = jnp.where(better, minv8, minv_ref[...])
    grp_ref[...] = jnp.where(better, j * NG + gbest, grp_ref[...])

    @pl.when(j == NB - 1)
    def _emit():
        minv = minv_ref[...]
        srow = jax.lax.broadcasted_iota(jnp.int32, (8, Q), 0)
        rid = grp_ref[...] * 8 + srow
        gmin = jnp.min(minv, axis=0, keepdims=True)
        cand = jnp.where(minv == gmin, rid, K)
        gidx = jnp.min(cand, axis=0, keepdims=True)
        dist_out[...] = jnp.maximum(gmin, 0.0)
        idx_out[...] = gidx


@jax.jit
def kernel(x, index):
    x_sq = jnp.sum(x * x, axis=1)[None, :]
    dist, idx = pl.pallas_call(
        _nn_kernel,
        grid=(NB,),
        in_specs=[
            pl.BlockSpec((Q, D), lambda j: (0, 0)),
            pl.BlockSpec((BK, D), lambda j: (j, 0)),
            pl.BlockSpec((1, Q), lambda j: (0, 0)),
        ],
        out_specs=[
            pl.BlockSpec((1, Q), lambda j: (0, 0)),
            pl.BlockSpec((1, Q), lambda j: (0, 0)),
        ],
        out_shape=[
            jax.ShapeDtypeStruct((1, Q), jnp.float32),
            jax.ShapeDtypeStruct((1, Q), jnp.int32),
        ],
        scratch_shapes=[
            pltpu.VMEM((8, Q), jnp.float32),
            pltpu.VMEM((8, Q), jnp.int32),
        ],
        compiler_params=pltpu.CompilerParams(
            dimension_semantics=("arbitrary",),
        ),
    )(x, index, x_sq)
    return dist.reshape(Q, 1), idx.reshape(Q, 1)


# speedup vs baseline: 8.4479x; 1.0515x over previous
import jax
import jax.numpy as jnp
from jax.experimental import pallas as pl
from jax.experimental.pallas import tpu as pltpu

Q = 1024
K = 100000
D = 64
BK = 1000
NB = K // BK
NG = BK // 8


def _nn_kernel(xm2_ref, idx_ref, xsq_ref, dist_out, idx_out,
               minv_ref, grp_ref):
    j = pl.program_id(0)

    @pl.when(j == 0)
    def _init():
        minv_ref[...] = jnp.full((8, Q), jnp.inf, jnp.float32)
        grp_ref[...] = jnp.zeros((8, Q), jnp.int32)

    blk = idx_ref[...]
    m2 = jax.lax.dot_general(
        blk, xm2_ref[...], (((1,), (1,)), ((), ())),
        preferred_element_type=jnp.float32)

    isq = jnp.sum(blk * blk, axis=1, keepdims=True)
    t = (xsq_ref[...] + isq) + m2

    minv8 = t[0:8, :]
    gbest = jnp.zeros((8, Q), jnp.int32)
    for g in range(1, NG):
        tg = t[g * 8:(g + 1) * 8, :]
        better = tg < minv8
        minv8 = jnp.where(better, tg, minv8)
        gbest = jnp.where(better, g, gbest)

    better = minv8 < minv_ref[...]
    minv_ref[...] = jnp.where(better, minv8, minv_ref[...])
    grp_ref[...] = jnp.where(better, j * NG + gbest, grp_ref[...])

    @pl.when(j == NB - 1)
    def _emit():
        minv = minv_ref[...]
        srow = jax.lax.broadcasted_iota(jnp.int32, (8, Q), 0)
        rid = grp_ref[...] * 8 + srow
        gmin = jnp.min(minv, axis=0, keepdims=True)
        cand = jnp.where(minv == gmin, rid, K)
        gidx = jnp.min(cand, axis=0, keepdims=True)
        dist_out[...] = jnp.maximum(gmin, 0.0)
        idx_out[...] = gidx


@jax.jit
def kernel(x, index):
    x_sq = jnp.sum(x * x, axis=1)[None, :]
    x_m2 = x * jnp.float32(-2.0)
    dist, idx = pl.pallas_call(
        _nn_kernel,
        grid=(NB,),
        in_specs=[
            pl.BlockSpec((Q, D), lambda j: (0, 0)),
            pl.BlockSpec((BK, D), lambda j: (j, 0)),
            pl.BlockSpec((1, Q), lambda j: (0, 0)),
        ],
        out_specs=[
            pl.BlockSpec((1, Q), lambda j: (0, 0)),
            pl.BlockSpec((1, Q), lambda j: (0, 0)),
        ],
        out_shape=[
            jax.ShapeDtypeStruct((1, Q), jnp.float32),
            jax.ShapeDtypeStruct((1, Q), jnp.int32),
        ],
        scratch_shapes=[
            pltpu.VMEM((8, Q), jnp.float32),
            pltpu.VMEM((8, Q), jnp.int32),
        ],
        compiler_params=pltpu.CompilerParams(
            dimension_semantics=("arbitrary",),
        ),
    )(x_m2, index, x_sq)
    return dist.reshape(Q, 1), idx.reshape(Q, 1)
